# baseline (speedup 1.0000x reference)
import functools

import jax
import jax.numpy as jnp
from jax import lax
from jax.experimental import pallas as pl
from jax.experimental.pallas import tpu as pltpu
from jax.experimental.pallas import tpu_sc as plsc

VOCAB = 1000000
EMBED = 64
BATCH = 4096
HIST = 200
N = BATCH * HIST
PADW = 128

_info = plsc.get_sparse_core_info()
NC = _info.num_cores
NS = _info.num_subcores
NW = NC * NS
BPW = N // NW
K = 256
NCHUNK = BPW // K
NRING = 3
LEAD = 2

_mesh = plsc.VectorSubcoreMesh(core_axis_name="c", subcore_axis_name="s")


@functools.partial(
    pl.kernel,
    mesh=_mesh,
    out_type=jax.ShapeDtypeStruct((N, PADW), jnp.float32),
    compiler_params=pltpu.CompilerParams(use_tc_tiling_on_sc=True),
    scratch_types=[pltpu.VMEM((BPW,), jnp.int32),
                   pltpu.VMEM((NRING, K, PADW), jnp.float32)]
                  + [pltpu.SemaphoreType.DMA] * NRING,
)
def _embed(seq_hbm, table_hbm, out_hbm, idx_v, rows_v, *sems):
    wid = lax.axis_index("s") * NC + lax.axis_index("c")
    base = wid * BPW
    pltpu.sync_copy(seq_hbm.at[pl.ds(base, BPW)], idx_v)

    def issue_gather(g, b):
        pltpu.async_copy(
            table_hbm.at[idx_v.at[pl.ds(g * K, K)]], rows_v.at[b], sems[b]
        )

    def wait_gather(b):
        pltpu.make_async_copy(
            table_hbm.at[pl.ds(0, K)], rows_v.at[b], sems[b]
        ).wait()

    def store(g, b):
        pltpu.sync_copy(rows_v.at[b], out_hbm.at[pl.ds(base + g * K, K)])

    for b in range(LEAD):
        issue_gather(b, b)

    def body(j, _):
        for i in range(NRING):
            g = j * NRING + i
            issue_gather(g + LEAD, (i + LEAD) % NRING)
            wait_gather(i)
            store(g, i)
        return 0

    lax.fori_loop(0, NCHUNK // NRING - 1, body, 0)

    done = (NCHUNK // NRING - 1) * NRING
    for g in range(done, NCHUNK):
        if g + LEAD < NCHUNK:
            issue_gather(g + LEAD, (g + LEAD) % NRING)
        wait_gather(g % NRING)
        store(g, g % NRING)


TBLK = 2048


def _tpose_body(wt_ref, out_ref):
    eye = (lax.broadcasted_iota(jnp.int32, (EMBED, PADW), 0)
           == lax.broadcasted_iota(jnp.int32, (EMBED, PADW), 1)
           ).astype(jnp.float32)
    out_ref[...] = lax.dot_general(
        wt_ref[...], eye,
        dimension_numbers=(((0,), (0,)), ((), ())),
        preferred_element_type=jnp.float32,
    )


_tpose = pl.pallas_call(
    _tpose_body,
    grid=((VOCAB + TBLK - 1) // TBLK,),
    in_specs=[pl.BlockSpec((EMBED, TBLK), lambda i: (0, i))],
    out_specs=pl.BlockSpec((TBLK, PADW), lambda i: (i, 0)),
    out_shape=jax.ShapeDtypeStruct((VOCAB, PADW), jnp.float32),
)


def kernel(seq, weight):
    wpad = _tpose(weight.T)
    out = _embed(seq.reshape(N), wpad)
    return out[:, :EMBED].reshape(BATCH, HIST, EMBED)


# speedup vs baseline: 1.2883x; 1.2883x over previous
import functools

import jax
import jax.numpy as jnp
from jax import lax
from jax.experimental import pallas as pl
from jax.experimental.pallas import tpu as pltpu
from jax.experimental.pallas import tpu_sc as plsc

VOCAB = 1000000
EMBED = 64
BATCH = 4096
HIST = 200
N = BATCH * HIST
PADW = 128

_info = plsc.get_sparse_core_info()
NC = _info.num_cores
NS = _info.num_subcores
NW = NC * NS
BPW = N // NW
K = 256
NCHUNK = BPW // K
NRING = 3
LEAD = 2

_mesh = plsc.VectorSubcoreMesh(core_axis_name="c", subcore_axis_name="s")


@functools.partial(
    pl.kernel,
    mesh=_mesh,
    out_type=jax.ShapeDtypeStruct((N, PADW), jnp.float32),
    compiler_params=pltpu.CompilerParams(use_tc_tiling_on_sc=True),
    scratch_types=[pltpu.VMEM((BPW,), jnp.int32),
                   pltpu.VMEM((NRING, K, PADW), jnp.float32)]
                  + [pltpu.SemaphoreType.DMA] * NRING,
)
def _embed(seq_hbm, table_hbm, out_hbm, idx_v, rows_v, *sems):
    wid = lax.axis_index("s") * NC + lax.axis_index("c")
    base = wid * BPW
    pltpu.sync_copy(seq_hbm.at[pl.ds(base, BPW)], idx_v)

    def issue_gather(g, b):
        pltpu.async_copy(
            table_hbm.at[idx_v.at[pl.ds(g * K, K)]], rows_v.at[b], sems[b]
        )

    def wait_gather(b):
        pltpu.make_async_copy(
            table_hbm.at[pl.ds(0, K)], rows_v.at[b], sems[b]
        ).wait()

    def store(g, b):
        pltpu.sync_copy(rows_v.at[b], out_hbm.at[pl.ds(base + g * K, K)])

    for b in range(LEAD):
        issue_gather(b, b)

    def body(j, _):
        for i in range(NRING):
            g = j * NRING + i
            issue_gather(g + LEAD, (i + LEAD) % NRING)
            wait_gather(i)
            store(g, i)
        return 0

    lax.fori_loop(0, NCHUNK // NRING - 1, body, 0)

    done = (NCHUNK // NRING - 1) * NRING
    for g in range(done, NCHUNK):
        if g + LEAD < NCHUNK:
            issue_gather(g + LEAD, (g + LEAD) % NRING)
        wait_gather(g % NRING)
        store(g, g % NRING)


TBLK = 8192


def _tpose_body(wt_ref, out_ref):
    out_ref[:, 0:EMBED] = wt_ref[...].T
    out_ref[:, EMBED:PADW] = jnp.zeros((TBLK, PADW - EMBED), jnp.float32)


_tpose = pl.pallas_call(
    _tpose_body,
    grid=((VOCAB + TBLK - 1) // TBLK,),
    in_specs=[pl.BlockSpec((EMBED, TBLK), lambda i: (0, i))],
    out_specs=pl.BlockSpec((TBLK, PADW), lambda i: (i, 0)),
    out_shape=jax.ShapeDtypeStruct((VOCAB, PADW), jnp.float32),
)


def kernel(seq, weight):
    wpad = _tpose(weight.T)
    out = _embed(seq.reshape(N), wpad)
    return out[:, :EMBED].reshape(BATCH, HIST, EMBED)


# speedup vs baseline: 1.3271x; 1.0301x over previous
import functools

import jax
import jax.numpy as jnp
from jax import lax
from jax.experimental import pallas as pl
from jax.experimental.pallas import tpu as pltpu
from jax.experimental.pallas import tpu_sc as plsc

VOCAB = 1000000
EMBED = 64
BATCH = 4096
HIST = 200
N = BATCH * HIST
PADW = 128

_info = plsc.get_sparse_core_info()
NC = _info.num_cores
NS = _info.num_subcores
NW = NC * NS
BPW = N // NW
K = 256
NCHUNK = BPW // K
NRING = 3
LEAD = 2

_mesh = plsc.VectorSubcoreMesh(core_axis_name="c", subcore_axis_name="s")


@functools.partial(
    pl.kernel,
    mesh=_mesh,
    out_type=jax.ShapeDtypeStruct((N, PADW), jnp.float32),
    compiler_params=pltpu.CompilerParams(use_tc_tiling_on_sc=True),
    scratch_types=[pltpu.VMEM((BPW,), jnp.int32),
                   pltpu.VMEM((NRING, K, PADW), jnp.float32)]
                  + [pltpu.SemaphoreType.DMA] * NRING,
)
def _embed(seq_hbm, table_hbm, out_hbm, idx_v, rows_v, *sems):
    wid = lax.axis_index("s") * NC + lax.axis_index("c")
    base = wid * BPW
    pltpu.sync_copy(seq_hbm.at[pl.ds(base, BPW)], idx_v)

    def issue_gather(g, b):
        pltpu.async_copy(
            table_hbm.at[idx_v.at[pl.ds(g * K, K)]], rows_v.at[b], sems[b]
        )

    def wait_gather(b):
        pltpu.make_async_copy(
            table_hbm.at[pl.ds(0, K)], rows_v.at[b], sems[b]
        ).wait()

    def store(g, b):
        pltpu.sync_copy(rows_v.at[b], out_hbm.at[pl.ds(base + g * K, K)])

    for b in range(LEAD):
        issue_gather(b, b)

    def body(j, _):
        for i in range(NRING):
            g = j * NRING + i
            issue_gather(g + LEAD, (i + LEAD) % NRING)
            wait_gather(i)
            store(g, i)
        return 0

    lax.fori_loop(0, NCHUNK // NRING - 1, body, 0)

    done = (NCHUNK // NRING - 1) * NRING
    for g in range(done, NCHUNK):
        if g + LEAD < NCHUNK:
            issue_gather(g + LEAD, (g + LEAD) % NRING)
        wait_gather(g % NRING)
        store(g, g % NRING)


TBLK = 16384


def _tpose_body(wt_ref, out_ref):
    out_ref[:, 0:EMBED] = wt_ref[...].T
    out_ref[:, EMBED:PADW] = jnp.zeros((TBLK, PADW - EMBED), jnp.float32)


_tpose = pl.pallas_call(
    _tpose_body,
    grid=((VOCAB + TBLK - 1) // TBLK,),
    in_specs=[pl.BlockSpec((EMBED, TBLK), lambda i: (0, i))],
    out_specs=pl.BlockSpec((TBLK, PADW), lambda i: (i, 0)),
    out_shape=jax.ShapeDtypeStruct((VOCAB, PADW), jnp.float32),
)


def kernel(seq, weight):
    wpad = _tpose(weight.T)
    out = _embed(seq.reshape(N), wpad)
    return out[:, :EMBED].reshape(BATCH, HIST, EMBED)


# speedup vs baseline: 1.3363x; 1.0069x over previous
import functools

import jax
import jax.numpy as jnp
from jax import lax
from jax.experimental import pallas as pl
from jax.experimental.pallas import tpu as pltpu
from jax.experimental.pallas import tpu_sc as plsc

VOCAB = 1000000
EMBED = 64
BATCH = 4096
HIST = 200
N = BATCH * HIST
PADW = 128

_info = plsc.get_sparse_core_info()
NC = _info.num_cores
NS = _info.num_subcores
NW = NC * NS
BPW = N // NW
K = 256
NCHUNK = BPW // K
NRING = 3
LEAD = 2

_mesh = plsc.VectorSubcoreMesh(core_axis_name="c", subcore_axis_name="s")


@functools.partial(
    pl.kernel,
    mesh=_mesh,
    out_type=jax.ShapeDtypeStruct((N, PADW), jnp.float32),
    compiler_params=pltpu.CompilerParams(use_tc_tiling_on_sc=True),
    scratch_types=[pltpu.VMEM((BPW,), jnp.int32),
                   pltpu.VMEM((NRING, K, PADW), jnp.float32)]
                  + [pltpu.SemaphoreType.DMA] * NRING,
)
def _embed(seq_hbm, table_hbm, out_hbm, idx_v, rows_v, *sems):
    wid = lax.axis_index("s") * NC + lax.axis_index("c")
    base = wid * BPW
    pltpu.sync_copy(seq_hbm.at[pl.ds(base, BPW)], idx_v)

    def issue_gather(g, b):
        pltpu.async_copy(
            table_hbm.at[idx_v.at[pl.ds(g * K, K)]], rows_v.at[b], sems[b]
        )

    def wait_gather(b):
        pltpu.make_async_copy(
            table_hbm.at[pl.ds(0, K)], rows_v.at[b], sems[b]
        ).wait()

    def store(g, b):
        pltpu.sync_copy(rows_v.at[b], out_hbm.at[pl.ds(base + g * K, K)])

    for b in range(LEAD):
        issue_gather(b, b)

    def body(j, _):
        for i in range(NRING):
            g = j * NRING + i
            issue_gather(g + LEAD, (i + LEAD) % NRING)
            wait_gather(i)
            store(g, i)
        return 0

    lax.fori_loop(0, NCHUNK // NRING - 1, body, 0)

    done = (NCHUNK // NRING - 1) * NRING
    for g in range(done, NCHUNK):
        if g + LEAD < NCHUNK:
            issue_gather(g + LEAD, (g + LEAD) % NRING)
        wait_gather(g % NRING)
        store(g, g % NRING)


TBLK = 32768


def _tpose_body(wt_ref, out_ref):
    out_ref[:, 0:EMBED] = wt_ref[...].T


_tpose = pl.pallas_call(
    _tpose_body,
    grid=((VOCAB + TBLK - 1) // TBLK,),
    in_specs=[pl.BlockSpec((EMBED, TBLK), lambda i: (0, i))],
    out_specs=pl.BlockSpec((TBLK, PADW), lambda i: (i, 0)),
    out_shape=jax.ShapeDtypeStruct((VOCAB, PADW), jnp.float32),
)


def kernel(seq, weight):
    wpad = _tpose(weight.T)
    out = _embed(seq.reshape(N), wpad)
    return out[:, :EMBED].reshape(BATCH, HIST, EMBED)
